# baseline (speedup 1.0000x reference)
import jax
import jax.numpy as jnp
from jax import lax
from jax.experimental import pallas as pl
from jax.experimental.pallas import tpu as pltpu
from jax.experimental.pallas import tpu_sc as plsc

N_NODES = 10000
D = 128
N_PAD = 10008
E_EDGES = 320000
NW = 32
CS = 128
CH = 79
TE = CS * CH
E_PAD = NW * TE
ROWS_PER_TILE = 632
NBUF = 3
IDEPTH = 4
KD = 8
DEG_W = 128


def _sc_agg_body(h_hbm, srcg_hbm, dstg_hbm, z128_hbm, acc_out,
                 src_ring, dst_ring, msg, acc_sh, isem, gsem, ssem):
    c = lax.axis_index("c")
    s = lax.axis_index("s")
    wid = s * 2 + c

    row0 = lax.min(s * ROWS_PER_TILE, N_PAD - ROWS_PER_TILE)
    pltpu.sync_copy(z128_hbm.at[pl.ds(row0, ROWS_PER_TILE)],
                    acc_sh.at[pl.ds(row0, ROWS_PER_TILE)])

    plsc.subcore_barrier()

    def mslice(j):
        return msg.at[pl.ds(lax.rem(j, NBUF) * CS, CS)]

    def sslice(j):
        return src_ring.at[pl.ds(lax.rem(j, IDEPTH) * CS, CS)]

    def dslice(j):
        return dst_ring.at[lax.rem(j, IDEPTH)]

    def idx_fetch(j):
        pltpu.async_copy(srcg_hbm.at[wid, j], sslice(j), isem)
        pltpu.async_copy(dstg_hbm.at[wid, j], dslice(j), isem)

    iwait_s = pltpu.make_async_copy(srcg_hbm.at[0, 0],
                                    src_ring.at[pl.ds(0, CS)], isem)
    iwait_d = pltpu.make_async_copy(dstg_hbm.at[0, 0], dst_ring.at[0], isem)
    gwait = pltpu.make_async_copy(h_hbm.at[src_ring.at[pl.ds(0, CS)]],
                                  msg.at[pl.ds(0, CS)], gsem)
    swait = pltpu.make_async_copy(msg.at[pl.ds(0, CS)],
                                  acc_sh.at[dst_ring.at[0]], ssem)

    def iwait_pair():
        iwait_s.wait()
        iwait_d.wait()

    def prologue(b, _):
        idx_fetch(b)
        return 0

    lax.fori_loop(0, IDEPTH - 1, prologue, 0)
    iwait_pair()
    pltpu.async_copy(h_hbm.at[sslice(0)], mslice(0), gsem)
    iwait_pair()
    pltpu.async_copy(h_hbm.at[sslice(1)], mslice(1), gsem)

    def body(j, _):
        @pl.when(j > 0)
        def _():
            swait.wait()

        @pl.when(j + IDEPTH - 1 < CH)
        def _():
            idx_fetch(j + IDEPTH - 1)

        @pl.when(j + 2 < CH)
        def _():
            iwait_pair()
            pltpu.async_copy(h_hbm.at[sslice(j + 2)], mslice(j + 2), gsem)

        gwait.wait()
        pltpu.async_copy(mslice(j), acc_sh.at[dslice(j)], ssem, add=True)
        return 0

    lax.fori_loop(0, CH, body, 0)
    swait.wait()

    plsc.subcore_barrier()

    pltpu.sync_copy(acc_sh.at[pl.ds(row0, ROWS_PER_TILE)],
                    acc_out.at[c, pl.ds(row0, ROWS_PER_TILE)])


_sc_agg = pl.kernel(
    _sc_agg_body,
    out_type=jax.ShapeDtypeStruct((2, N_PAD, D), jnp.float32),
    mesh=plsc.VectorSubcoreMesh(core_axis_name="c", subcore_axis_name="s"),
    scratch_types=[
        pltpu.VMEM((IDEPTH * CS,), jnp.int32),
        pltpu.VMEM((IDEPTH, CS), jnp.int32),
        pltpu.VMEM((NBUF * CS, D), jnp.float32),
        pltpu.VMEM_SHARED((N_PAD, D), jnp.float32),
        pltpu.SemaphoreType.DMA,
        pltpu.SemaphoreType.DMA,
        pltpu.SemaphoreType.DMA,
    ],
)


def _sc_deg_body(dstg_hbm, z128_hbm, ones_hbm, deg_out, dst_idx, ones_v,
                 deg_sh, dsem):
    c = lax.axis_index("c")
    s = lax.axis_index("s")
    wid = s * 2 + c

    row0 = lax.min(s * ROWS_PER_TILE, N_PAD - ROWS_PER_TILE)
    pltpu.sync_copy(z128_hbm.at[pl.ds(row0, ROWS_PER_TILE)],
                    deg_sh.at[pl.ds(row0, ROWS_PER_TILE)])
    pltpu.sync_copy(dstg_hbm.at[wid], dst_idx)
    pltpu.sync_copy(ones_hbm, ones_v)

    plsc.subcore_barrier()

    dwait = pltpu.make_async_copy(ones_v, deg_sh.at[dst_idx.at[0]], dsem)

    def body(j, _):
        @pl.when(j >= KD)
        def _():
            dwait.wait()
        pltpu.async_copy(ones_v, deg_sh.at[dst_idx.at[j]], dsem, add=True)
        return 0

    lax.fori_loop(0, CH, body, 0)

    def drain(j, _):
        dwait.wait()
        return 0

    lax.fori_loop(0, min(KD, CH), drain, 0)

    plsc.subcore_barrier()

    pltpu.sync_copy(deg_sh.at[pl.ds(row0, ROWS_PER_TILE)],
                    deg_out.at[c, pl.ds(row0, ROWS_PER_TILE)])


_sc_deg = pl.kernel(
    _sc_deg_body,
    out_type=jax.ShapeDtypeStruct((2, N_PAD, DEG_W), jnp.float32),
    mesh=plsc.VectorSubcoreMesh(core_axis_name="c", subcore_axis_name="s"),
    scratch_types=[
        pltpu.VMEM((CH, CS), jnp.int32),
        pltpu.VMEM((CS, DEG_W), jnp.float32),
        pltpu.VMEM_SHARED((N_PAD, DEG_W), jnp.float32),
        pltpu.SemaphoreType.DMA,
    ],
)

ROW_BLK = 1000


def _tc_layer_body(acc_ref, deg_ref, x_ref, wl_ref, bl_ref, wr_ref, out_ref):
    a = acc_ref[0] + acc_ref[1]
    d = deg_ref[0, :, 0:1] + deg_ref[1, :, 0:1]
    agg = a / jnp.maximum(d, 1.0)
    z = (jnp.dot(agg, wl_ref[...], preferred_element_type=jnp.float32)
         + bl_ref[...]
         + jnp.dot(x_ref[...], wr_ref[...], preferred_element_type=jnp.float32))
    out_ref[...] = jnp.maximum(z, 0.0)


def _tc_final_body(acc_ref, deg_ref, x_ref, wl_ref, bl_ref, wr_ref, wo_ref,
                   bo_ref, out_ref):
    a = acc_ref[0] + acc_ref[1]
    d = deg_ref[0, :, 0:1] + deg_ref[1, :, 0:1]
    agg = a / jnp.maximum(d, 1.0)
    z = (jnp.dot(agg, wl_ref[...], preferred_element_type=jnp.float32)
         + bl_ref[...]
         + jnp.dot(x_ref[...], wr_ref[...], preferred_element_type=jnp.float32))
    h = jnp.maximum(z, 0.0)
    out_ref[...] = (jnp.dot(h, wo_ref[...], preferred_element_type=jnp.float32)
                    + bo_ref[...])


def _tc_layer(acc, deg, x, wl, bl, wr):
    grid = N_NODES // ROW_BLK
    return pl.pallas_call(
        _tc_layer_body,
        grid=(grid,),
        in_specs=[
            pl.BlockSpec((2, ROW_BLK, D), lambda i: (0, i, 0)),
            pl.BlockSpec((2, ROW_BLK, DEG_W), lambda i: (0, i, 0)),
            pl.BlockSpec((ROW_BLK, D), lambda i: (i, 0)),
            pl.BlockSpec((D, D), lambda i: (0, 0)),
            pl.BlockSpec((1, D), lambda i: (0, 0)),
            pl.BlockSpec((D, D), lambda i: (0, 0)),
        ],
        out_specs=pl.BlockSpec((ROW_BLK, D), lambda i: (i, 0)),
        out_shape=jax.ShapeDtypeStruct((N_NODES, D), jnp.float32),
    )(acc, deg, x, wl, bl.reshape(1, D), wr)


def _tc_final(acc, deg, x, wl, bl, wr, wo, bo):
    grid = N_NODES // ROW_BLK
    return pl.pallas_call(
        _tc_final_body,
        grid=(grid,),
        in_specs=[
            pl.BlockSpec((2, ROW_BLK, D), lambda i: (0, i, 0)),
            pl.BlockSpec((2, ROW_BLK, DEG_W), lambda i: (0, i, 0)),
            pl.BlockSpec((ROW_BLK, D), lambda i: (i, 0)),
            pl.BlockSpec((D, D), lambda i: (0, 0)),
            pl.BlockSpec((1, D), lambda i: (0, 0)),
            pl.BlockSpec((D, D), lambda i: (0, 0)),
            pl.BlockSpec((D, 1), lambda i: (0, 0)),
            pl.BlockSpec((1, 1), lambda i: (0, 0)),
        ],
        out_specs=pl.BlockSpec((ROW_BLK, 1), lambda i: (i, 0)),
        out_shape=jax.ShapeDtypeStruct((N_NODES, 1), jnp.float32),
    )(acc, deg, x, wl, bl.reshape(1, D), wr, wo, bo.reshape(1, 1))


def kernel(x, edge_index, Wl1, bl1, Wr1, Wl2, bl2, Wr2, Wo, bo):
    src = edge_index[0].astype(jnp.int32)
    dst = edge_index[1].astype(jnp.int32)
    pad = E_PAD - E_EDGES
    srcg = jnp.concatenate([src, jnp.zeros((pad,), jnp.int32)]).reshape(
        NW, CH, CS)
    dstg = jnp.concatenate(
        [dst, jnp.full((pad,), N_NODES, jnp.int32)]).reshape(NW, CH, CS)
    z128 = jnp.zeros((N_PAD, D), jnp.float32)

    ones16 = jnp.ones((CS, DEG_W), jnp.float32)
    deg = _sc_deg(dstg, z128, ones16)
    acc1 = _sc_agg(x, srcg, dstg, z128)
    h1 = _tc_layer(acc1, deg, x, Wl1, bl1, Wr1)
    acc2 = _sc_agg(h1, srcg, dstg, z128)
    out = _tc_final(acc2, deg, h1, Wl2, bl2, Wr2, Wo, bo)
    return out[:, 0]


# speedup vs baseline: 1.0048x; 1.0048x over previous
import jax
import jax.numpy as jnp
from jax import lax
from jax.experimental import pallas as pl
from jax.experimental.pallas import tpu as pltpu
from jax.experimental.pallas import tpu_sc as plsc

N_NODES = 10000
D = 128
N_PAD = 10008
E_EDGES = 320000
NW = 32
CS = 128
CH = 79
TE = CS * CH
E_PAD = NW * TE
CH0 = 79
CH1 = 79
CHM = max(CH0, CH1)
ROWS_PER_TILE = 632
NBUF = 3
IDEPTH = 4
KD = 8
DEG_W = 128


def _sc_agg_body(h_hbm, srcg_hbm, dstg_hbm, z128_hbm, acc_out,
                 src_ring, dst_ring, msg, acc_sh, isem, gsem, ssem):
    c = lax.axis_index("c")
    s = lax.axis_index("s")
    wid = s * 2 + c
    chw = jnp.where(c == 0, CH0, CH1)

    row0 = lax.min(s * ROWS_PER_TILE, N_PAD - ROWS_PER_TILE)
    pltpu.sync_copy(z128_hbm.at[pl.ds(row0, ROWS_PER_TILE)],
                    acc_sh.at[pl.ds(row0, ROWS_PER_TILE)])

    plsc.subcore_barrier()

    def mslice(j):
        return msg.at[pl.ds(lax.rem(j, NBUF) * CS, CS)]

    def sslice(j):
        return src_ring.at[pl.ds(lax.rem(j, IDEPTH) * CS, CS)]

    def dslice(j):
        return dst_ring.at[lax.rem(j, IDEPTH)]

    def idx_fetch(j):
        pltpu.async_copy(srcg_hbm.at[wid, j], sslice(j), isem)
        pltpu.async_copy(dstg_hbm.at[wid, j], dslice(j), isem)

    iwait_s = pltpu.make_async_copy(srcg_hbm.at[0, 0],
                                    src_ring.at[pl.ds(0, CS)], isem)
    iwait_d = pltpu.make_async_copy(dstg_hbm.at[0, 0], dst_ring.at[0], isem)
    gwait = pltpu.make_async_copy(h_hbm.at[src_ring.at[pl.ds(0, CS)]],
                                  msg.at[pl.ds(0, CS)], gsem)
    swait = pltpu.make_async_copy(msg.at[pl.ds(0, CS)],
                                  acc_sh.at[dst_ring.at[0]], ssem)

    def iwait_pair():
        iwait_s.wait()
        iwait_d.wait()

    def prologue(b, _):
        idx_fetch(b)
        return 0

    lax.fori_loop(0, IDEPTH - 1, prologue, 0)
    iwait_pair()
    pltpu.async_copy(h_hbm.at[sslice(0)], mslice(0), gsem)
    iwait_pair()
    pltpu.async_copy(h_hbm.at[sslice(1)], mslice(1), gsem)

    def body(j, _):
        @pl.when(j > 0)
        def _():
            swait.wait()

        @pl.when(j + IDEPTH - 1 < chw)
        def _():
            idx_fetch(j + IDEPTH - 1)

        @pl.when(j + 2 < chw)
        def _():
            iwait_pair()
            pltpu.async_copy(h_hbm.at[sslice(j + 2)], mslice(j + 2), gsem)

        gwait.wait()
        pltpu.async_copy(mslice(j), acc_sh.at[dslice(j)], ssem, add=True)
        return 0

    lax.fori_loop(0, chw, body, 0)
    swait.wait()

    plsc.subcore_barrier()

    pltpu.sync_copy(acc_sh.at[pl.ds(row0, ROWS_PER_TILE)],
                    acc_out.at[c, pl.ds(row0, ROWS_PER_TILE)])


_sc_agg = pl.kernel(
    _sc_agg_body,
    out_type=jax.ShapeDtypeStruct((2, N_PAD, D), jnp.float32),
    mesh=plsc.VectorSubcoreMesh(core_axis_name="c", subcore_axis_name="s"),
    scratch_types=[
        pltpu.VMEM((IDEPTH * CS,), jnp.int32),
        pltpu.VMEM((IDEPTH, CS), jnp.int32),
        pltpu.VMEM((NBUF * CS, D), jnp.float32),
        pltpu.VMEM_SHARED((N_PAD, D), jnp.float32),
        pltpu.SemaphoreType.DMA,
        pltpu.SemaphoreType.DMA,
        pltpu.SemaphoreType.DMA,
    ],
)


def _sc_deg_body(dstg_hbm, z128_hbm, ones_hbm, deg_out, dst_idx, ones_v,
                 deg_sh, dsem):
    c = lax.axis_index("c")
    s = lax.axis_index("s")
    wid = s * 2 + c

    row0 = lax.min(s * ROWS_PER_TILE, N_PAD - ROWS_PER_TILE)
    pltpu.sync_copy(z128_hbm.at[pl.ds(row0, ROWS_PER_TILE)],
                    deg_sh.at[pl.ds(row0, ROWS_PER_TILE)])
    pltpu.sync_copy(dstg_hbm.at[wid], dst_idx)
    pltpu.sync_copy(ones_hbm, ones_v)

    plsc.subcore_barrier()

    dwait = pltpu.make_async_copy(ones_v, deg_sh.at[dst_idx.at[0]], dsem)

    def body(j, _):
        @pl.when(j >= KD)
        def _():
            dwait.wait()
        pltpu.async_copy(ones_v, deg_sh.at[dst_idx.at[j]], dsem, add=True)
        return 0

    lax.fori_loop(0, CH, body, 0)

    def drain(j, _):
        dwait.wait()
        return 0

    lax.fori_loop(0, min(KD, CH), drain, 0)

    plsc.subcore_barrier()

    pltpu.sync_copy(deg_sh.at[pl.ds(row0, ROWS_PER_TILE)],
                    deg_out.at[c, pl.ds(row0, ROWS_PER_TILE)])


_sc_deg = pl.kernel(
    _sc_deg_body,
    out_type=jax.ShapeDtypeStruct((2, N_PAD, DEG_W), jnp.float32),
    mesh=plsc.VectorSubcoreMesh(core_axis_name="c", subcore_axis_name="s"),
    scratch_types=[
        pltpu.VMEM((CH, CS), jnp.int32),
        pltpu.VMEM((CS, DEG_W), jnp.float32),
        pltpu.VMEM_SHARED((N_PAD, DEG_W), jnp.float32),
        pltpu.SemaphoreType.DMA,
    ],
)

ROW_BLK = 1000


def _tc_layer_body(acc_ref, deg_ref, x_ref, wl_ref, bl_ref, wr_ref, out_ref):
    a = acc_ref[0] + acc_ref[1]
    d = deg_ref[0, :, 0:1] + deg_ref[1, :, 0:1]
    agg = a / jnp.maximum(d, 1.0)
    z = (jnp.dot(agg, wl_ref[...], preferred_element_type=jnp.float32)
         + bl_ref[...]
         + jnp.dot(x_ref[...], wr_ref[...], preferred_element_type=jnp.float32))
    out_ref[...] = jnp.maximum(z, 0.0)


def _tc_final_body(acc_ref, deg_ref, x_ref, wl_ref, bl_ref, wr_ref, wo_ref,
                   bo_ref, out_ref):
    a = acc_ref[0] + acc_ref[1]
    d = deg_ref[0, :, 0:1] + deg_ref[1, :, 0:1]
    agg = a / jnp.maximum(d, 1.0)
    z = (jnp.dot(agg, wl_ref[...], preferred_element_type=jnp.float32)
         + bl_ref[...]
         + jnp.dot(x_ref[...], wr_ref[...], preferred_element_type=jnp.float32))
    h = jnp.maximum(z, 0.0)
    out_ref[...] = (jnp.dot(h, wo_ref[...], preferred_element_type=jnp.float32)
                    + bo_ref[...])


def _tc_layer(acc, deg, x, wl, bl, wr):
    grid = N_NODES // ROW_BLK
    return pl.pallas_call(
        _tc_layer_body,
        grid=(grid,),
        in_specs=[
            pl.BlockSpec((2, ROW_BLK, D), lambda i: (0, i, 0)),
            pl.BlockSpec((2, ROW_BLK, DEG_W), lambda i: (0, i, 0)),
            pl.BlockSpec((ROW_BLK, D), lambda i: (i, 0)),
            pl.BlockSpec((D, D), lambda i: (0, 0)),
            pl.BlockSpec((1, D), lambda i: (0, 0)),
            pl.BlockSpec((D, D), lambda i: (0, 0)),
        ],
        out_specs=pl.BlockSpec((ROW_BLK, D), lambda i: (i, 0)),
        out_shape=jax.ShapeDtypeStruct((N_NODES, D), jnp.float32),
    )(acc, deg, x, wl, bl.reshape(1, D), wr)


def _tc_final(acc, deg, x, wl, bl, wr, wo, bo):
    grid = N_NODES // ROW_BLK
    return pl.pallas_call(
        _tc_final_body,
        grid=(grid,),
        in_specs=[
            pl.BlockSpec((2, ROW_BLK, D), lambda i: (0, i, 0)),
            pl.BlockSpec((2, ROW_BLK, DEG_W), lambda i: (0, i, 0)),
            pl.BlockSpec((ROW_BLK, D), lambda i: (i, 0)),
            pl.BlockSpec((D, D), lambda i: (0, 0)),
            pl.BlockSpec((1, D), lambda i: (0, 0)),
            pl.BlockSpec((D, D), lambda i: (0, 0)),
            pl.BlockSpec((D, 1), lambda i: (0, 0)),
            pl.BlockSpec((1, 1), lambda i: (0, 0)),
        ],
        out_specs=pl.BlockSpec((ROW_BLK, 1), lambda i: (i, 0)),
        out_shape=jax.ShapeDtypeStruct((N_NODES, 1), jnp.float32),
    )(acc, deg, x, wl, bl.reshape(1, D), wr, wo, bo.reshape(1, 1))


def _pack_agg(v, fill):
    lens = [CS * (CH0 if (w % 2) == 0 else CH1) for w in range(NW)]
    offs = [0]
    for n in lens:
        offs.append(offs[-1] + n)
    rows = []
    for w in range(NW):
        seg = v[offs[w]:offs[w + 1]]
        if lens[w] < CHM * CS:
            seg = jnp.concatenate(
                [seg, jnp.full((CHM * CS - lens[w],), fill, jnp.int32)])
        rows.append(seg)
    return jnp.stack(rows).reshape(NW, CHM, CS)


def kernel(x, edge_index, Wl1, bl1, Wr1, Wl2, bl2, Wr2, Wo, bo):
    src = edge_index[0].astype(jnp.int32)
    dst = edge_index[1].astype(jnp.int32)
    pad = E_PAD - E_EDGES
    srcp = jnp.concatenate([src, jnp.zeros((pad,), jnp.int32)])
    dstp = jnp.concatenate([dst, jnp.full((pad,), N_NODES, jnp.int32)])
    srcg = _pack_agg(srcp, 0)
    dstg = _pack_agg(dstp, N_NODES)
    dstg_deg = dstp.reshape(NW, CH, CS)
    z128 = jnp.zeros((N_PAD, D), jnp.float32)

    ones16 = jnp.ones((CS, DEG_W), jnp.float32)
    deg = _sc_deg(dstg_deg, z128, ones16)
    acc1 = _sc_agg(x, srcg, dstg, z128)
    h1 = _tc_layer(acc1, deg, x, Wl1, bl1, Wr1)
    acc2 = _sc_agg(h1, srcg, dstg, z128)
    out = _tc_final(acc2, deg, h1, Wl2, bl2, Wr2, Wo, bo)
    return out[:, 0]


# speedup vs baseline: 1.1536x; 1.1481x over previous
import jax
import jax.numpy as jnp
from jax import lax
from jax.experimental import pallas as pl
from jax.experimental.pallas import tpu as pltpu
from jax.experimental.pallas import tpu_sc as plsc

N_NODES = 10000
D = 128
N_PAD = 10008
E_EDGES = 320000
NW = 32
CS = 128
CH = 79
TE = CS * CH
E_PAD = NW * TE
CH0 = 110
CH1 = 48
CHM = max(CH0, CH1)
ROWS_PER_TILE = 632
NBUF = 3
IDEPTH = 4
KD = 8
DEG_W = 128


def _sc_agg_body(h_hbm, srcg_hbm, dstg_hbm, z128_hbm, acc_out,
                 src_ring, dst_ring, msg, acc_sh, isem, gsem, ssem):
    c = lax.axis_index("c")
    s = lax.axis_index("s")
    wid = s * 2 + c
    chw = jnp.where(c == 0, CH0, CH1)

    row0 = lax.min(s * ROWS_PER_TILE, N_PAD - ROWS_PER_TILE)
    pltpu.sync_copy(z128_hbm.at[pl.ds(row0, ROWS_PER_TILE)],
                    acc_sh.at[pl.ds(row0, ROWS_PER_TILE)])

    plsc.subcore_barrier()

    def mslice(j):
        return msg.at[pl.ds(lax.rem(j, NBUF) * CS, CS)]

    def sslice(j):
        return src_ring.at[pl.ds(lax.rem(j, IDEPTH) * CS, CS)]

    def dslice(j):
        return dst_ring.at[lax.rem(j, IDEPTH)]

    def idx_fetch(j):
        pltpu.async_copy(srcg_hbm.at[wid, j], sslice(j), isem)
        pltpu.async_copy(dstg_hbm.at[wid, j], dslice(j), isem)

    iwait_s = pltpu.make_async_copy(srcg_hbm.at[0, 0],
                                    src_ring.at[pl.ds(0, CS)], isem)
    iwait_d = pltpu.make_async_copy(dstg_hbm.at[0, 0], dst_ring.at[0], isem)
    gwait = pltpu.make_async_copy(h_hbm.at[src_ring.at[pl.ds(0, CS)]],
                                  msg.at[pl.ds(0, CS)], gsem)
    swait = pltpu.make_async_copy(msg.at[pl.ds(0, CS)],
                                  acc_sh.at[dst_ring.at[0]], ssem)

    def iwait_pair():
        iwait_s.wait()
        iwait_d.wait()

    def prologue(b, _):
        idx_fetch(b)
        return 0

    lax.fori_loop(0, IDEPTH - 1, prologue, 0)
    iwait_pair()
    pltpu.async_copy(h_hbm.at[sslice(0)], mslice(0), gsem)
    iwait_pair()
    pltpu.async_copy(h_hbm.at[sslice(1)], mslice(1), gsem)

    def body(j, _):
        @pl.when(j > 0)
        def _():
            swait.wait()

        @pl.when(j + IDEPTH - 1 < chw)
        def _():
            idx_fetch(j + IDEPTH - 1)

        @pl.when(j + 2 < chw)
        def _():
            iwait_pair()
            pltpu.async_copy(h_hbm.at[sslice(j + 2)], mslice(j + 2), gsem)

        gwait.wait()
        pltpu.async_copy(mslice(j), acc_sh.at[dslice(j)], ssem, add=True)
        return 0

    lax.fori_loop(0, chw, body, 0)
    swait.wait()

    plsc.subcore_barrier()

    pltpu.sync_copy(acc_sh.at[pl.ds(row0, ROWS_PER_TILE)],
                    acc_out.at[c, pl.ds(row0, ROWS_PER_TILE)])


_sc_agg = pl.kernel(
    _sc_agg_body,
    out_type=jax.ShapeDtypeStruct((2, N_PAD, D), jnp.float32),
    mesh=plsc.VectorSubcoreMesh(core_axis_name="c", subcore_axis_name="s"),
    scratch_types=[
        pltpu.VMEM((IDEPTH * CS,), jnp.int32),
        pltpu.VMEM((IDEPTH, CS), jnp.int32),
        pltpu.VMEM((NBUF * CS, D), jnp.float32),
        pltpu.VMEM_SHARED((N_PAD, D), jnp.float32),
        pltpu.SemaphoreType.DMA,
        pltpu.SemaphoreType.DMA,
        pltpu.SemaphoreType.DMA,
    ],
)


def _sc_deg_body(dstg_hbm, z128_hbm, ones_hbm, deg_out, dst_idx, ones_v,
                 deg_sh, dsem):
    c = lax.axis_index("c")
    s = lax.axis_index("s")
    wid = s * 2 + c

    row0 = lax.min(s * ROWS_PER_TILE, N_PAD - ROWS_PER_TILE)
    pltpu.sync_copy(z128_hbm.at[pl.ds(row0, ROWS_PER_TILE)],
                    deg_sh.at[pl.ds(row0, ROWS_PER_TILE)])
    pltpu.sync_copy(dstg_hbm.at[wid], dst_idx)
    pltpu.sync_copy(ones_hbm, ones_v)

    plsc.subcore_barrier()

    dwait = pltpu.make_async_copy(ones_v, deg_sh.at[dst_idx.at[0]], dsem)

    def body(j, _):
        @pl.when(j >= KD)
        def _():
            dwait.wait()
        pltpu.async_copy(ones_v, deg_sh.at[dst_idx.at[j]], dsem, add=True)
        return 0

    lax.fori_loop(0, CH, body, 0)

    def drain(j, _):
        dwait.wait()
        return 0

    lax.fori_loop(0, min(KD, CH), drain, 0)

    plsc.subcore_barrier()

    pltpu.sync_copy(deg_sh.at[pl.ds(row0, ROWS_PER_TILE)],
                    deg_out.at[c, pl.ds(row0, ROWS_PER_TILE)])


_sc_deg = pl.kernel(
    _sc_deg_body,
    out_type=jax.ShapeDtypeStruct((2, N_PAD, DEG_W), jnp.float32),
    mesh=plsc.VectorSubcoreMesh(core_axis_name="c", subcore_axis_name="s"),
    scratch_types=[
        pltpu.VMEM((CH, CS), jnp.int32),
        pltpu.VMEM((CS, DEG_W), jnp.float32),
        pltpu.VMEM_SHARED((N_PAD, DEG_W), jnp.float32),
        pltpu.SemaphoreType.DMA,
    ],
)

ROW_BLK = 1000


def _tc_layer_body(acc_ref, deg_ref, x_ref, wl_ref, bl_ref, wr_ref, out_ref):
    a = acc_ref[0] + acc_ref[1]
    d = deg_ref[0, :, 0:1] + deg_ref[1, :, 0:1]
    agg = a / jnp.maximum(d, 1.0)
    z = (jnp.dot(agg, wl_ref[...], preferred_element_type=jnp.float32)
         + bl_ref[...]
         + jnp.dot(x_ref[...], wr_ref[...], preferred_element_type=jnp.float32))
    out_ref[...] = jnp.maximum(z, 0.0)


def _tc_final_body(acc_ref, deg_ref, x_ref, wl_ref, bl_ref, wr_ref, wo_ref,
                   bo_ref, out_ref):
    a = acc_ref[0] + acc_ref[1]
    d = deg_ref[0, :, 0:1] + deg_ref[1, :, 0:1]
    agg = a / jnp.maximum(d, 1.0)
    z = (jnp.dot(agg, wl_ref[...], preferred_element_type=jnp.float32)
         + bl_ref[...]
         + jnp.dot(x_ref[...], wr_ref[...], preferred_element_type=jnp.float32))
    h = jnp.maximum(z, 0.0)
    out_ref[...] = (jnp.dot(h, wo_ref[...], preferred_element_type=jnp.float32)
                    + bo_ref[...])


def _tc_layer(acc, deg, x, wl, bl, wr):
    grid = N_NODES // ROW_BLK
    return pl.pallas_call(
        _tc_layer_body,
        grid=(grid,),
        in_specs=[
            pl.BlockSpec((2, ROW_BLK, D), lambda i: (0, i, 0)),
            pl.BlockSpec((2, ROW_BLK, DEG_W), lambda i: (0, i, 0)),
            pl.BlockSpec((ROW_BLK, D), lambda i: (i, 0)),
            pl.BlockSpec((D, D), lambda i: (0, 0)),
            pl.BlockSpec((1, D), lambda i: (0, 0)),
            pl.BlockSpec((D, D), lambda i: (0, 0)),
        ],
        out_specs=pl.BlockSpec((ROW_BLK, D), lambda i: (i, 0)),
        out_shape=jax.ShapeDtypeStruct((N_NODES, D), jnp.float32),
    )(acc, deg, x, wl, bl.reshape(1, D), wr)


def _tc_final(acc, deg, x, wl, bl, wr, wo, bo):
    grid = N_NODES // ROW_BLK
    return pl.pallas_call(
        _tc_final_body,
        grid=(grid,),
        in_specs=[
            pl.BlockSpec((2, ROW_BLK, D), lambda i: (0, i, 0)),
            pl.BlockSpec((2, ROW_BLK, DEG_W), lambda i: (0, i, 0)),
            pl.BlockSpec((ROW_BLK, D), lambda i: (i, 0)),
            pl.BlockSpec((D, D), lambda i: (0, 0)),
            pl.BlockSpec((1, D), lambda i: (0, 0)),
            pl.BlockSpec((D, D), lambda i: (0, 0)),
            pl.BlockSpec((D, 1), lambda i: (0, 0)),
            pl.BlockSpec((1, 1), lambda i: (0, 0)),
        ],
        out_specs=pl.BlockSpec((ROW_BLK, 1), lambda i: (i, 0)),
        out_shape=jax.ShapeDtypeStruct((N_NODES, 1), jnp.float32),
    )(acc, deg, x, wl, bl.reshape(1, D), wr, wo, bo.reshape(1, 1))


def _pack_agg(v, fill):
    lens = [CS * (CH0 if (w % 2) == 0 else CH1) for w in range(NW)]
    offs = [0]
    for n in lens:
        offs.append(offs[-1] + n)
    rows = []
    for w in range(NW):
        seg = v[offs[w]:offs[w + 1]]
        if lens[w] < CHM * CS:
            seg = jnp.concatenate(
                [seg, jnp.full((CHM * CS - lens[w],), fill, jnp.int32)])
        rows.append(seg)
    return jnp.stack(rows).reshape(NW, CHM, CS)


def kernel(x, edge_index, Wl1, bl1, Wr1, Wl2, bl2, Wr2, Wo, bo):
    src = edge_index[0].astype(jnp.int32)
    dst = edge_index[1].astype(jnp.int32)
    pad = E_PAD - E_EDGES
    srcp = jnp.concatenate([src, jnp.zeros((pad,), jnp.int32)])
    dstp = jnp.concatenate([dst, jnp.full((pad,), N_NODES, jnp.int32)])
    srcg = _pack_agg(srcp, 0)
    dstg = _pack_agg(dstp, N_NODES)
    dstg_deg = dstp.reshape(NW, CH, CS)
    z128 = jnp.zeros((N_PAD, D), jnp.float32)

    ones16 = jnp.ones((CS, DEG_W), jnp.float32)
    deg = _sc_deg(dstg_deg, z128, ones16)
    acc1 = _sc_agg(x, srcg, dstg, z128)
    h1 = _tc_layer(acc1, deg, x, Wl1, bl1, Wr1)
    acc2 = _sc_agg(h1, srcg, dstg, z128)
    out = _tc_final(acc2, deg, h1, Wl2, bl2, Wr2, Wo, bo)
    return out[:, 0]
